# baseline (speedup 1.0000x reference)
import jax
import jax.numpy as jnp
from jax import lax
from jax.experimental import pallas as pl
from jax.experimental.pallas import tpu as pltpu
from jax.experimental.pallas import tpu_sc as plsc

H = 128
N_NODES = 10000
LAYERS = 2
NC = 2
NS = 16
CHUNK = 128
N_CHUNKS = 80
NBUF = 4
N_PAD = 10240
ROWS_PER_TILE = N_PAD // NS
W_CNT = 128
BLK = 1000
NB = N_NODES // BLK
F32 = jnp.float32
HIGH = lax.Precision.HIGHEST


def _zero_acc_stripe(buf, acc_sh, s):
    w = buf.shape[1]

    def zero_body(i, _):
        for k in range(w // 16):
            buf[i, pl.ds(k * 16, 16)] = jnp.zeros((16,), F32)
        return 0
    lax.fori_loop(0, CHUNK, zero_body, 0)
    for k in range(ROWS_PER_TILE // CHUNK):
        pltpu.sync_copy(buf, acc_sh.at[pl.ds(s * ROWS_PER_TILE + k * CHUNK,
                                             CHUNK)])


def _sc_segsum_body(x_hbm, pk_hbm, out_hbm, acc_sh, pk_v, srcb, dstb,
                    r0, r1, s0, s1, t0, t1):
    c = lax.axis_index("c")
    s = lax.axis_index("s")
    rows = (r0, r1)
    sems = (s0, s1)
    ssems = (t0, t1)

    _zero_acc_stripe(r0, acc_sh, s)
    plsc.subcore_barrier()

    pltpu.sync_copy(pk_hbm.at[c, s], pk_v)

    def unpack(j, k):
        for g in range(CHUNK // 16):
            pk = pk_v[j, pl.ds(g * 16, 16)]
            srcb[k, pl.ds(g * 16, 16)] = lax.shift_right_logical(pk, 14)
            dstb[k, pl.ds(g * 16, 16)] = lax.bitwise_and(pk, 16383)

    for k in range(2):
        unpack(k, k)
        pltpu.async_copy(x_hbm.at[srcb.at[k]], rows[k], sems[k])

    def body(jj, _):
        for k in range(2):
            pltpu.make_async_copy(x_hbm.at[srcb.at[k]], rows[k],
                                  sems[k]).wait()
            pltpu.async_copy(rows[k], acc_sh.at[dstb.at[k]], ssems[k],
                             add=True)
        for k in range(2):
            j = jj * 2 + k
            pltpu.make_async_copy(rows[k], acc_sh.at[dstb.at[k]],
                                  ssems[k]).wait()

            @pl.when(jj < N_CHUNKS // 2 - 1)
            def _():
                unpack(j + 2, k)
                pltpu.async_copy(x_hbm.at[srcb.at[k]], rows[k], sems[k])
        return 0

    lax.fori_loop(0, N_CHUNKS // 2, body, 0)
    plsc.subcore_barrier()

    base = s * ROWS_PER_TILE
    pltpu.sync_copy(acc_sh.at[pl.ds(base, ROWS_PER_TILE)],
                    out_hbm.at[c, pl.ds(base, ROWS_PER_TILE)])


def _sc_segsum(x, packed):
    mesh = plsc.VectorSubcoreMesh(core_axis_name="c", subcore_axis_name="s",
                                  num_cores=NC, num_subcores=NS)
    return pl.kernel(
        _sc_segsum_body,
        out_type=jax.ShapeDtypeStruct((NC, N_PAD, H), F32),
        mesh=mesh,
        scratch_types=[
            pltpu.VMEM_SHARED((N_PAD, H), F32),
            pltpu.VMEM((N_CHUNKS, CHUNK), jnp.int32),
            pltpu.VMEM((2, CHUNK), jnp.int32),
            pltpu.VMEM((2, CHUNK), jnp.int32),
            pltpu.VMEM((CHUNK, H), F32),
            pltpu.VMEM((CHUNK, H), F32),
            pltpu.SemaphoreType.DMA,
            pltpu.SemaphoreType.DMA,
            pltpu.SemaphoreType.DMA,
            pltpu.SemaphoreType.DMA,
        ],
    )(x, packed)


def _sc_counts_body(dst_hbm, out_hbm, acc_sh, dst_v, ones_v):
    c = lax.axis_index("c")
    s = lax.axis_index("s")

    _zero_acc_stripe(ones_v, acc_sh, s)

    def fill(i, _):
        for k in range(W_CNT // 16):
            ones_v[i, pl.ds(k * 16, 16)] = jnp.ones((16,), F32)
        return 0
    lax.fori_loop(0, CHUNK, fill, 0)
    plsc.subcore_barrier()

    pltpu.sync_copy(dst_hbm.at[c, s], dst_v)

    def body(j, _):
        pltpu.sync_copy(ones_v, acc_sh.at[dst_v.at[j]], add=True)
        return 0
    lax.fori_loop(0, N_CHUNKS, body, 0)
    plsc.subcore_barrier()

    base = s * ROWS_PER_TILE
    pltpu.sync_copy(acc_sh.at[pl.ds(base, ROWS_PER_TILE)],
                    out_hbm.at[c, pl.ds(base, ROWS_PER_TILE)])


def _sc_counts(dst):
    mesh = plsc.VectorSubcoreMesh(core_axis_name="c", subcore_axis_name="s",
                                  num_cores=NC, num_subcores=NS)
    return pl.kernel(
        _sc_counts_body,
        out_type=jax.ShapeDtypeStruct((NC, N_PAD, W_CNT), F32),
        mesh=mesh,
        scratch_types=[
            pltpu.VMEM_SHARED((N_PAD, W_CNT), F32),
            pltpu.VMEM((N_CHUNKS, CHUNK), jnp.int32),
            pltpu.VMEM((CHUNK, W_CNT), F32),
        ],
    )(dst)


def _prep_body(ws_ref, wd_ref, wu_ref, bs_ref, bd_ref, bu_ref,
               a_ref, b_ref, cv_ref):
    wu_top = wu_ref[0, 0, :H, :]
    wu_bot = wu_ref[0, 0, H:, :]
    a_ref[0, 0] = jnp.dot(wd_ref[0, 0], wu_top, precision=HIGH,
                          preferred_element_type=F32)
    b_ref[0, 0] = jnp.dot(ws_ref[0, 0], wu_bot, precision=HIGH,
                          preferred_element_type=F32)
    cv_ref[0, 0] = (jnp.dot(bd_ref[0, 0], wu_top, precision=HIGH,
                            preferred_element_type=F32)
                    + jnp.dot(bs_ref[0, 0], wu_bot, precision=HIGH,
                              preferred_element_type=F32)
                    + bu_ref[0, 0])


def _tc_prep(W_src, W_dst, W_upd, b_src, b_dst, b_upd):
    bs = b_src.reshape(LAYERS, 2, 1, H)
    bd = b_dst.reshape(LAYERS, 2, 1, H)
    bu = b_upd.reshape(LAYERS, 2, 1, H)
    m4 = lambda i, j: (i, j, 0, 0)
    return pl.pallas_call(
        _prep_body,
        grid=(LAYERS, 2),
        in_specs=[
            pl.BlockSpec((1, 1, H, H), m4),
            pl.BlockSpec((1, 1, H, H), m4),
            pl.BlockSpec((1, 1, 2 * H, H), m4),
            pl.BlockSpec((1, 1, 1, H), m4),
            pl.BlockSpec((1, 1, 1, H), m4),
            pl.BlockSpec((1, 1, 1, H), m4),
        ],
        out_specs=[
            pl.BlockSpec((1, 1, H, H), m4),
            pl.BlockSpec((1, 1, H, H), m4),
            pl.BlockSpec((1, 1, 1, H), m4),
        ],
        out_shape=[
            jax.ShapeDtypeStruct((LAYERS, 2, H, H), F32),
            jax.ShapeDtypeStruct((LAYERS, 2, H, H), F32),
            jax.ShapeDtypeStruct((LAYERS, 2, 1, H), F32),
        ],
    )(W_src, W_dst, W_upd, bs, bd, bu)


def _mm_body(x_ref, p_ref, c_ref, a_ref, b_ref, cv_ref, y_ref, st_ref):
    i = pl.program_id(1)
    cnt = c_ref[0, :, 0:1]
    recip = 1.0 / jnp.maximum(cnt, 1.0)
    agg = p_ref[0] * recip
    y = (jnp.dot(x_ref[...], a_ref[0], precision=HIGH,
                 preferred_element_type=F32)
         + jnp.dot(agg, b_ref[0], precision=HIGH,
                   preferred_element_type=F32)
         + cv_ref[0])
    y_ref[...] = y

    @pl.when(i == 0)
    def _():
        st_ref[...] = jnp.zeros_like(st_ref)

    sums = jnp.sum(y, axis=0)[None, :]
    sumsq = jnp.sum(y * y, axis=0)[None, :]
    st_ref[...] += jnp.concatenate(
        [sums, sumsq, jnp.zeros((6, H), F32)], axis=0)[None]


def _tc_matmul_stats(X, p, cnt, A_l, cv_l):
    A, B = A_l
    return pl.pallas_call(
        _mm_body,
        grid=(2, NB),
        in_specs=[
            pl.BlockSpec((BLK, H), lambda t, i: (t * NB + i, 0)),
            pl.BlockSpec((1, BLK, H), lambda t, i: (t, i, 0)),
            pl.BlockSpec((1, BLK, W_CNT), lambda t, i: (t, i, 0)),
            pl.BlockSpec((1, H, H), lambda t, i: (t, 0, 0)),
            pl.BlockSpec((1, H, H), lambda t, i: (t, 0, 0)),
            pl.BlockSpec((1, 1, H), lambda t, i: (t, 0, 0)),
        ],
        out_specs=[
            pl.BlockSpec((BLK, H), lambda t, i: (t * NB + i, 0)),
            pl.BlockSpec((1, 8, H), lambda t, i: (t, 0, 0)),
        ],
        out_shape=[
            jax.ShapeDtypeStruct((2 * N_NODES, H), F32),
            jax.ShapeDtypeStruct((2, 8, H), F32),
        ],
    )(X, p, cnt, A, B, cv_l)


def _bn_lrelu(y_ref, st_ref, g_ref, b_ref):
    n = jnp.float32(N_NODES)
    m = st_ref[0, 0:1, :] / n
    v = st_ref[0, 1:2, :] / n - m * m
    scale = g_ref[0] / jnp.sqrt(v + 1.0)
    t = (y_ref[...] - m) * scale + b_ref[0]
    return jnp.where(t >= 0, t, 0.01 * t)


def _norm_body(y_ref, st_ref, g_ref, b_ref, o_ref):
    o_ref[...] = _bn_lrelu(y_ref, st_ref, g_ref, b_ref)


def _tc_norm(y, st, gamma, beta):
    return pl.pallas_call(
        _norm_body,
        grid=(2, NB),
        in_specs=[
            pl.BlockSpec((BLK, H), lambda t, i: (t * NB + i, 0)),
            pl.BlockSpec((1, 8, H), lambda t, i: (t, 0, 0)),
            pl.BlockSpec((1, 1, H), lambda t, i: (t, 0, 0)),
            pl.BlockSpec((1, 1, H), lambda t, i: (t, 0, 0)),
        ],
        out_specs=pl.BlockSpec((BLK, H), lambda t, i: (t * NB + i, 0)),
        out_shape=jax.ShapeDtypeStruct((2 * N_NODES, H), F32),
    )(y, st, gamma, beta)


def _norm_fc_body(y_ref, st_ref, g_ref, b_ref, w_ref, fb_ref, o_ref):
    xn = _bn_lrelu(y_ref, st_ref, g_ref, b_ref)
    o_ref[...] = jnp.dot(xn, w_ref[0], precision=HIGH,
                         preferred_element_type=F32) + fb_ref[0]


def _tc_norm_fc(y, st, gamma, beta, fw, fb):
    return pl.pallas_call(
        _norm_fc_body,
        grid=(2, NB),
        in_specs=[
            pl.BlockSpec((BLK, H), lambda t, i: (t * NB + i, 0)),
            pl.BlockSpec((1, 8, H), lambda t, i: (t, 0, 0)),
            pl.BlockSpec((1, 1, H), lambda t, i: (t, 0, 0)),
            pl.BlockSpec((1, 1, H), lambda t, i: (t, 0, 0)),
            pl.BlockSpec((1, H, 1), lambda t, i: (t, 0, 0)),
            pl.BlockSpec((1, 1, 1), lambda t, i: (t, 0, 0)),
        ],
        out_specs=pl.BlockSpec((BLK, 1), lambda t, i: (t * NB + i, 0)),
        out_shape=jax.ShapeDtypeStruct((2 * N_NODES, 1), F32),
    )(y, st, gamma, beta, fw, fb.reshape(2, 1, 1))


def _prep_edges(ei, src_off):
    e = ei.shape[1]
    e_pad = NS * N_CHUNKS * CHUNK
    npad = e_pad - e
    ar = jnp.arange(npad, dtype=jnp.int32)
    src = jnp.concatenate([ei[0].astype(jnp.int32) + src_off,
                           ar % (2 * N_NODES)])
    dst = jnp.concatenate([ei[1].astype(jnp.int32),
                           N_NODES + ar % (N_PAD - N_NODES)])
    shape = (NS, N_CHUNKS, CHUNK)
    return (src * 16384 + dst).reshape(shape), dst.reshape(shape)


def kernel(x_user, x_item, edge_index_ui, edge_index_iu, W_src, b_src,
           W_dst, b_dst, W_upd, b_upd, bn_gamma, bn_beta, fc_W, fc_b):
    pk0, d0 = _prep_edges(edge_index_ui, N_NODES)
    pk1, d1 = _prep_edges(edge_index_iu, 0)
    packed = jnp.stack([pk0, pk1])
    dst = jnp.stack([d0, d1])

    cnt = _sc_counts(dst)
    A, B, cv = _tc_prep(W_src, W_dst, W_upd, b_src, b_dst, b_upd)
    gam = bn_gamma[:, ::-1].reshape(LAYERS, 2, 1, H)
    bet = bn_beta[:, ::-1].reshape(LAYERS, 2, 1, H)

    X = jnp.concatenate([x_item, x_user], axis=0)
    out = None
    for i in range(LAYERS):
        p = _sc_segsum(X, packed)
        y, st = _tc_matmul_stats(X, p, cnt, (A[i], B[i]), cv[i])
        if i < LAYERS - 1:
            X = _tc_norm(y, st, gam[i], bet[i])
        else:
            out = _tc_norm_fc(y, st, gam[i], bet[i], fc_W[::-1], fc_b[::-1])
    return (out[N_NODES:], out[:N_NODES])


# speedup vs baseline: 1.1503x; 1.1503x over previous
import jax
import jax.numpy as jnp
from jax import lax
from jax.experimental import pallas as pl
from jax.experimental.pallas import tpu as pltpu
from jax.experimental.pallas import tpu_sc as plsc

H = 128
N_NODES = 10000
LAYERS = 2
NC = 2
NS = 16
CHUNK = 128
N_CHUNKS = 80
NBUF = 4
N_PAD = 10240
ROWS_PER_TILE = N_PAD // NS
W_CNT = 128
BLK = 1000
NB = N_NODES // BLK
F32 = jnp.float32
HIGH = lax.Precision.HIGHEST


def _zero_acc_stripe(buf, acc_sh, s):
    w = buf.shape[1]

    def zero_body(i, _):
        for k in range(w // 16):
            buf[i, pl.ds(k * 16, 16)] = jnp.zeros((16,), F32)
        return 0
    lax.fori_loop(0, CHUNK, zero_body, 0)
    for k in range(ROWS_PER_TILE // CHUNK):
        pltpu.sync_copy(buf, acc_sh.at[pl.ds(s * ROWS_PER_TILE + k * CHUNK,
                                             CHUNK)])


def _sc_segsum_body(x_hbm, pk_hbm, out_hbm, acc_sh, pk_v, srcb, dstb,
                    r0, r1, s0, s1):
    c = lax.axis_index("c")
    s = lax.axis_index("s")
    rows = (r0, r1)
    sems = (s0, s1)

    _zero_acc_stripe(r0, acc_sh, s)
    plsc.subcore_barrier()

    pltpu.sync_copy(pk_hbm.at[c, s], pk_v)

    def unpack(j, k):
        for g in range(CHUNK // 16):
            pk = pk_v[j, pl.ds(g * 16, 16)]
            srcb[k, pl.ds(g * 16, 16)] = lax.shift_right_logical(pk, 14)
            dstb[k, pl.ds(g * 16, 16)] = lax.bitwise_and(pk, 16383)

    for k in range(2):
        unpack(k, k)
        pltpu.async_copy(x_hbm.at[srcb.at[k]], rows[k], sems[k])

    def body(jj, _):
        for k in range(2):
            j = jj * 2 + k
            pltpu.make_async_copy(x_hbm.at[srcb.at[k]], rows[k],
                                  sems[k]).wait()
            pltpu.sync_copy(rows[k], acc_sh.at[dstb.at[k]], add=True)

            @pl.when(jj < N_CHUNKS // 2 - 1)
            def _():
                unpack(j + 2, k)
                pltpu.async_copy(x_hbm.at[srcb.at[k]], rows[k], sems[k])
        return 0

    lax.fori_loop(0, N_CHUNKS // 2, body, 0)
    plsc.subcore_barrier()

    base = s * ROWS_PER_TILE
    pltpu.sync_copy(acc_sh.at[pl.ds(base, ROWS_PER_TILE)],
                    out_hbm.at[c, pl.ds(base, ROWS_PER_TILE)])


def _sc_segsum(x, packed):
    mesh = plsc.VectorSubcoreMesh(core_axis_name="c", subcore_axis_name="s",
                                  num_cores=NC, num_subcores=NS)
    return pl.kernel(
        _sc_segsum_body,
        out_type=jax.ShapeDtypeStruct((NC, N_PAD, H), F32),
        mesh=mesh,
        scratch_types=[
            pltpu.VMEM_SHARED((N_PAD, H), F32),
            pltpu.VMEM((N_CHUNKS, CHUNK), jnp.int32),
            pltpu.VMEM((2, CHUNK), jnp.int32),
            pltpu.VMEM((2, CHUNK), jnp.int32),
            pltpu.VMEM((CHUNK, H), F32),
            pltpu.VMEM((CHUNK, H), F32),
            pltpu.SemaphoreType.DMA,
            pltpu.SemaphoreType.DMA,
        ],
    )(x, packed)


def _sc_counts_body(dst_hbm, out_hbm, acc_sh, dst_v, ones_v):
    c = lax.axis_index("c")
    s = lax.axis_index("s")

    _zero_acc_stripe(ones_v, acc_sh, s)

    def fill(i, _):
        for k in range(W_CNT // 16):
            ones_v[i, pl.ds(k * 16, 16)] = jnp.ones((16,), F32)
        return 0
    lax.fori_loop(0, CHUNK, fill, 0)
    plsc.subcore_barrier()

    pltpu.sync_copy(dst_hbm.at[c, s], dst_v)

    def body(j, _):
        pltpu.sync_copy(ones_v, acc_sh.at[dst_v.at[j]], add=True)
        return 0
    lax.fori_loop(0, N_CHUNKS, body, 0)
    plsc.subcore_barrier()

    base = s * ROWS_PER_TILE
    pltpu.sync_copy(acc_sh.at[pl.ds(base, ROWS_PER_TILE)],
                    out_hbm.at[c, pl.ds(base, ROWS_PER_TILE)])


def _sc_counts(dst):
    mesh = plsc.VectorSubcoreMesh(core_axis_name="c", subcore_axis_name="s",
                                  num_cores=NC, num_subcores=NS)
    return pl.kernel(
        _sc_counts_body,
        out_type=jax.ShapeDtypeStruct((NC, N_PAD, W_CNT), F32),
        mesh=mesh,
        scratch_types=[
            pltpu.VMEM_SHARED((N_PAD, W_CNT), F32),
            pltpu.VMEM((N_CHUNKS, CHUNK), jnp.int32),
            pltpu.VMEM((CHUNK, W_CNT), F32),
        ],
    )(dst)


def _prep_body(ws_ref, wd_ref, wu_ref, bs_ref, bd_ref, bu_ref,
               a_ref, b_ref, cv_ref):
    wu_top = wu_ref[0, 0, :H, :]
    wu_bot = wu_ref[0, 0, H:, :]
    a_ref[0, 0] = jnp.dot(wd_ref[0, 0], wu_top, precision=HIGH,
                          preferred_element_type=F32)
    b_ref[0, 0] = jnp.dot(ws_ref[0, 0], wu_bot, precision=HIGH,
                          preferred_element_type=F32)
    cv_ref[0, 0] = (jnp.dot(bd_ref[0, 0], wu_top, precision=HIGH,
                            preferred_element_type=F32)
                    + jnp.dot(bs_ref[0, 0], wu_bot, precision=HIGH,
                              preferred_element_type=F32)
                    + bu_ref[0, 0])


def _tc_prep(W_src, W_dst, W_upd, b_src, b_dst, b_upd):
    bs = b_src.reshape(LAYERS, 2, 1, H)
    bd = b_dst.reshape(LAYERS, 2, 1, H)
    bu = b_upd.reshape(LAYERS, 2, 1, H)
    m4 = lambda i, j: (i, j, 0, 0)
    return pl.pallas_call(
        _prep_body,
        grid=(LAYERS, 2),
        in_specs=[
            pl.BlockSpec((1, 1, H, H), m4),
            pl.BlockSpec((1, 1, H, H), m4),
            pl.BlockSpec((1, 1, 2 * H, H), m4),
            pl.BlockSpec((1, 1, 1, H), m4),
            pl.BlockSpec((1, 1, 1, H), m4),
            pl.BlockSpec((1, 1, 1, H), m4),
        ],
        out_specs=[
            pl.BlockSpec((1, 1, H, H), m4),
            pl.BlockSpec((1, 1, H, H), m4),
            pl.BlockSpec((1, 1, 1, H), m4),
        ],
        out_shape=[
            jax.ShapeDtypeStruct((LAYERS, 2, H, H), F32),
            jax.ShapeDtypeStruct((LAYERS, 2, H, H), F32),
            jax.ShapeDtypeStruct((LAYERS, 2, 1, H), F32),
        ],
    )(W_src, W_dst, W_upd, bs, bd, bu)


def _layer_compute(x_ref, p_ref, c_ref, a_ref, b_ref, cv_ref):
    cnt = c_ref[0, :, 0:1]
    recip = 1.0 / jnp.maximum(cnt, 1.0)
    agg = p_ref[0] * recip
    return (jnp.dot(x_ref[...], a_ref[0], precision=HIGH,
                    preferred_element_type=F32)
            + jnp.dot(agg, b_ref[0], precision=HIGH,
                      preferred_element_type=F32)
            + cv_ref[0])


def _layer_phase0(i, y, y_scr, st_scr):
    y_scr[pl.ds(i * BLK, BLK), :] = y

    @pl.when(i == 0)
    def _():
        st_scr[...] = jnp.zeros_like(st_scr)

    sums = jnp.sum(y, axis=0)[None, :]
    sumsq = jnp.sum(y * y, axis=0)[None, :]
    st_scr[...] += jnp.concatenate(
        [sums, sumsq, jnp.zeros((6, H), F32)], axis=0)


def _bn_lrelu(i, y_scr, st_scr, g_ref, b_ref):
    n = jnp.float32(N_NODES)
    m = st_scr[0:1, :] / n
    v = st_scr[1:2, :] / n - m * m
    scale = g_ref[0] / jnp.sqrt(v + 1.0)
    t = (y_scr[pl.ds(i * BLK, BLK), :] - m) * scale + b_ref[0]
    return jnp.where(t >= 0, t, 0.01 * t)


def _layer_body(x_ref, p_ref, c_ref, a_ref, b_ref, cv_ref, g_ref, be_ref,
                o_ref, y_scr, st_scr):
    ph = pl.program_id(1)
    i = pl.program_id(2)

    @pl.when(ph == 0)
    def _():
        _layer_phase0(i, _layer_compute(x_ref, p_ref, c_ref, a_ref, b_ref,
                                        cv_ref), y_scr, st_scr)

    @pl.when(ph == 1)
    def _():
        o_ref[...] = _bn_lrelu(i, y_scr, st_scr, g_ref, be_ref)


def _layer_fc_body(x_ref, p_ref, c_ref, a_ref, b_ref, cv_ref, g_ref, be_ref,
                   w_ref, fb_ref, o_ref, y_scr, st_scr):
    ph = pl.program_id(1)
    i = pl.program_id(2)

    @pl.when(ph == 0)
    def _():
        _layer_phase0(i, _layer_compute(x_ref, p_ref, c_ref, a_ref, b_ref,
                                        cv_ref), y_scr, st_scr)

    @pl.when(ph == 1)
    def _():
        xn = _bn_lrelu(i, y_scr, st_scr, g_ref, be_ref)
        o_ref[...] = jnp.dot(xn, w_ref[0], precision=HIGH,
                             preferred_element_type=F32) + fb_ref[0]


def _layer_specs():
    return [
        pl.BlockSpec((BLK, H), lambda t, ph, i: (t * NB + i, 0)),
        pl.BlockSpec((1, BLK, H), lambda t, ph, i: (t, i, 0)),
        pl.BlockSpec((1, BLK, W_CNT), lambda t, ph, i: (t, i, 0)),
        pl.BlockSpec((1, H, H), lambda t, ph, i: (t, 0, 0)),
        pl.BlockSpec((1, H, H), lambda t, ph, i: (t, 0, 0)),
        pl.BlockSpec((1, 1, H), lambda t, ph, i: (t, 0, 0)),
        pl.BlockSpec((1, 1, H), lambda t, ph, i: (t, 0, 0)),
        pl.BlockSpec((1, 1, H), lambda t, ph, i: (t, 0, 0)),
    ]


def _out_map(w):
    return lambda t, ph, i: (ph * (t * NB + i) + (1 - ph) * 2 * NB, 0)


def _tc_layer(X, p, cnt, A_l, B_l, cv_l, gamma, beta):
    return pl.pallas_call(
        _layer_body,
        grid=(2, 2, NB),
        in_specs=_layer_specs(),
        out_specs=pl.BlockSpec((BLK, H), _out_map(H)),
        out_shape=jax.ShapeDtypeStruct((2 * N_NODES + BLK, H), F32),
        scratch_shapes=[
            pltpu.VMEM((N_NODES, H), F32),
            pltpu.VMEM((8, H), F32),
        ],
    )(X, p, cnt, A_l, B_l, cv_l, gamma, beta)


def _tc_layer_fc(X, p, cnt, A_l, B_l, cv_l, gamma, beta, fw, fb):
    return pl.pallas_call(
        _layer_fc_body,
        grid=(2, 2, NB),
        in_specs=_layer_specs() + [
            pl.BlockSpec((1, H, 1), lambda t, ph, i: (t, 0, 0)),
            pl.BlockSpec((1, 1, 1), lambda t, ph, i: (t, 0, 0)),
        ],
        out_specs=pl.BlockSpec((BLK, 1), _out_map(1)),
        out_shape=jax.ShapeDtypeStruct((2 * N_NODES + BLK, 1), F32),
        scratch_shapes=[
            pltpu.VMEM((N_NODES, H), F32),
            pltpu.VMEM((8, H), F32),
        ],
    )(X, p, cnt, A_l, B_l, cv_l, gamma, beta, fw, fb.reshape(2, 1, 1))


def _prep_edges(ei, src_off):
    e = ei.shape[1]
    e_pad = NS * N_CHUNKS * CHUNK
    npad = e_pad - e
    ar = jnp.arange(npad, dtype=jnp.int32)
    src = jnp.concatenate([ei[0].astype(jnp.int32) + src_off,
                           ar % (2 * N_NODES)])
    dst = jnp.concatenate([ei[1].astype(jnp.int32),
                           N_NODES + ar % (N_PAD - N_NODES)])
    shape = (NS, N_CHUNKS, CHUNK)
    return (src * 16384 + dst).reshape(shape), dst.reshape(shape)


def kernel(x_user, x_item, edge_index_ui, edge_index_iu, W_src, b_src,
           W_dst, b_dst, W_upd, b_upd, bn_gamma, bn_beta, fc_W, fc_b):
    pk0, d0 = _prep_edges(edge_index_ui, N_NODES)
    pk1, d1 = _prep_edges(edge_index_iu, 0)
    packed = jnp.stack([pk0, pk1])
    dst = jnp.stack([d0, d1])

    cnt = _sc_counts(dst)
    A, B, cv = _tc_prep(W_src, W_dst, W_upd, b_src, b_dst, b_upd)
    gam = bn_gamma[:, ::-1].reshape(LAYERS, 2, 1, H)
    bet = bn_beta[:, ::-1].reshape(LAYERS, 2, 1, H)

    X = jnp.concatenate([x_item, x_user], axis=0)
    out = None
    for i in range(LAYERS):
        p = _sc_segsum(X, packed)
        if i < LAYERS - 1:
            X = _tc_layer(X, p, cnt, A[i], B[i], cv[i], gam[i], bet[i])
        else:
            out = _tc_layer_fc(X, p, cnt, A[i], B[i], cv[i], gam[i],
                               bet[i], fc_W[::-1], fc_b[::-1])
    return (out[N_NODES:2 * N_NODES], out[:N_NODES])


# speedup vs baseline: 1.3059x; 1.1353x over previous
import jax
import jax.numpy as jnp
from jax import lax
from jax.experimental import pallas as pl
from jax.experimental.pallas import tpu as pltpu
from jax.experimental.pallas import tpu_sc as plsc

H = 128
N_NODES = 10000
LAYERS = 2
NC = 2
NS = 16
CHUNK = 128
N_CHUNKS = 80
NBUF = 4
N_PAD = 10240
ROWS_PER_TILE = N_PAD // NS
W_CNT = 128
BLK = 1000
NB = N_NODES // BLK
F32 = jnp.float32
HIGH = lax.Precision.HIGHEST


def _zero_acc_stripe(buf, acc_sh, s):
    w = buf.shape[1]

    def zero_body(i, _):
        for k in range(w // 16):
            buf[i, pl.ds(k * 16, 16)] = jnp.zeros((16,), F32)
        return 0
    lax.fori_loop(0, CHUNK, zero_body, 0)
    for k in range(ROWS_PER_TILE // CHUNK):
        pltpu.sync_copy(buf, acc_sh.at[pl.ds(s * ROWS_PER_TILE + k * CHUNK,
                                             CHUNK)])


def _sc_segsum_body(x_hbm, pk_hbm, out_hbm, acc_sh, pk_v, srcb, dstb,
                    r0, r1, s0, s1):
    c = lax.axis_index("c")
    s = lax.axis_index("s")
    rows = (r0, r1)
    sems = (s0, s1)

    _zero_acc_stripe(r0, acc_sh, s)
    plsc.subcore_barrier()

    pltpu.sync_copy(pk_hbm.at[c, s], pk_v)

    def unpack(j, k):
        for g in range(CHUNK // 16):
            pk = pk_v[j, pl.ds(g * 16, 16)]
            srcb[k, pl.ds(g * 16, 16)] = lax.shift_right_logical(pk, 14)
            dstb[k, pl.ds(g * 16, 16)] = lax.bitwise_and(pk, 16383)

    for k in range(2):
        unpack(k, k)
        pltpu.async_copy(x_hbm.at[srcb.at[k]], rows[k], sems[k])

    def body(jj, _):
        for k in range(2):
            j = jj * 2 + k
            pltpu.make_async_copy(x_hbm.at[srcb.at[k]], rows[k],
                                  sems[k]).wait()
            pltpu.sync_copy(rows[k], acc_sh.at[dstb.at[k]], add=True)

            @pl.when(jj < N_CHUNKS // 2 - 1)
            def _():
                unpack(j + 2, k)
                pltpu.async_copy(x_hbm.at[srcb.at[k]], rows[k], sems[k])
        return 0

    lax.fori_loop(0, N_CHUNKS // 2, body, 0)
    plsc.subcore_barrier()

    base = s * ROWS_PER_TILE
    pltpu.sync_copy(acc_sh.at[pl.ds(base, ROWS_PER_TILE)],
                    out_hbm.at[c, pl.ds(base, ROWS_PER_TILE)])


def _sc_segsum(x, packed):
    mesh = plsc.VectorSubcoreMesh(core_axis_name="c", subcore_axis_name="s",
                                  num_cores=NC, num_subcores=NS)
    return pl.kernel(
        _sc_segsum_body,
        out_type=jax.ShapeDtypeStruct((NC, N_PAD, H), F32),
        mesh=mesh,
        scratch_types=[
            pltpu.VMEM_SHARED((N_PAD, H), F32),
            pltpu.VMEM((N_CHUNKS, CHUNK), jnp.int32),
            pltpu.VMEM((2, CHUNK), jnp.int32),
            pltpu.VMEM((2, CHUNK), jnp.int32),
            pltpu.VMEM((CHUNK, H), F32),
            pltpu.VMEM((CHUNK, H), F32),
            pltpu.SemaphoreType.DMA,
            pltpu.SemaphoreType.DMA,
        ],
    )(x, packed)


def _sc_counts_body(dst_hbm, out_hbm, acc_sh, dst_v, ones_v):
    c = lax.axis_index("c")
    s = lax.axis_index("s")

    _zero_acc_stripe(ones_v, acc_sh, s)

    def fill(i, _):
        for k in range(W_CNT // 16):
            ones_v[i, pl.ds(k * 16, 16)] = jnp.ones((16,), F32)
        return 0
    lax.fori_loop(0, CHUNK, fill, 0)
    plsc.subcore_barrier()

    pltpu.sync_copy(dst_hbm.at[c, s], dst_v)

    def body(j, _):
        pltpu.sync_copy(ones_v, acc_sh.at[dst_v.at[j]], add=True)
        return 0
    lax.fori_loop(0, N_CHUNKS, body, 0)
    plsc.subcore_barrier()

    base = s * ROWS_PER_TILE
    pltpu.sync_copy(acc_sh.at[pl.ds(base, ROWS_PER_TILE)],
                    out_hbm.at[c, pl.ds(base, ROWS_PER_TILE)])


def _sc_counts(dst):
    mesh = plsc.VectorSubcoreMesh(core_axis_name="c", subcore_axis_name="s",
                                  num_cores=NC, num_subcores=NS)
    return pl.kernel(
        _sc_counts_body,
        out_type=jax.ShapeDtypeStruct((NC, N_PAD, W_CNT), F32),
        mesh=mesh,
        scratch_types=[
            pltpu.VMEM_SHARED((N_PAD, W_CNT), F32),
            pltpu.VMEM((N_CHUNKS, CHUNK), jnp.int32),
            pltpu.VMEM((CHUNK, W_CNT), F32),
        ],
    )(dst)


def _prep_body(ws_ref, wd_ref, wu_ref, bs_ref, bd_ref, bu_ref,
               a_ref, b_ref, cv_ref):
    wu_top = wu_ref[0, 0, :H, :]
    wu_bot = wu_ref[0, 0, H:, :]
    a_ref[0, 0] = jnp.dot(wd_ref[0, 0], wu_top, precision=HIGH,
                          preferred_element_type=F32)
    b_ref[0, 0] = jnp.dot(ws_ref[0, 0], wu_bot, precision=HIGH,
                          preferred_element_type=F32)
    cv_ref[0, 0] = (jnp.dot(bd_ref[0, 0], wu_top, precision=HIGH,
                            preferred_element_type=F32)
                    + jnp.dot(bs_ref[0, 0], wu_bot, precision=HIGH,
                              preferred_element_type=F32)
                    + bu_ref[0, 0])


def _tc_prep(W_src, W_dst, W_upd, b_src, b_dst, b_upd):
    bs = b_src.reshape(LAYERS, 2, 1, H)
    bd = b_dst.reshape(LAYERS, 2, 1, H)
    bu = b_upd.reshape(LAYERS, 2, 1, H)
    m4 = lambda i, j: (i, j, 0, 0)
    return pl.pallas_call(
        _prep_body,
        grid=(LAYERS, 2),
        in_specs=[
            pl.BlockSpec((1, 1, H, H), m4),
            pl.BlockSpec((1, 1, H, H), m4),
            pl.BlockSpec((1, 1, 2 * H, H), m4),
            pl.BlockSpec((1, 1, 1, H), m4),
            pl.BlockSpec((1, 1, 1, H), m4),
            pl.BlockSpec((1, 1, 1, H), m4),
        ],
        out_specs=[
            pl.BlockSpec((1, 1, H, H), m4),
            pl.BlockSpec((1, 1, H, H), m4),
            pl.BlockSpec((1, 1, 1, H), m4),
        ],
        out_shape=[
            jax.ShapeDtypeStruct((LAYERS, 2, H, H), F32),
            jax.ShapeDtypeStruct((LAYERS, 2, H, H), F32),
            jax.ShapeDtypeStruct((LAYERS, 2, 1, H), F32),
        ],
    )(W_src, W_dst, W_upd, bs, bd, bu)


def _layer_compute(x_ref, p_ref, c_ref, a_ref, b_ref, cv_ref):
    cnt = c_ref[0, :, 0:1]
    recip = 1.0 / jnp.maximum(cnt, 1.0)
    agg = p_ref[0] * recip
    return (jnp.dot(x_ref[...], a_ref[0], preferred_element_type=F32)
            + jnp.dot(agg, b_ref[0], preferred_element_type=F32)
            + cv_ref[0])


def _layer_phase0(i, y, y_scr, st_scr):
    y_scr[pl.ds(i * BLK, BLK), :] = y

    @pl.when(i == 0)
    def _():
        st_scr[...] = jnp.zeros_like(st_scr)

    sums = jnp.sum(y, axis=0)[None, :]
    sumsq = jnp.sum(y * y, axis=0)[None, :]
    st_scr[...] += jnp.concatenate(
        [sums, sumsq, jnp.zeros((6, H), F32)], axis=0)


def _bn_lrelu(i, y_scr, st_scr, g_ref, b_ref):
    n = jnp.float32(N_NODES)
    m = st_scr[0:1, :] / n
    v = st_scr[1:2, :] / n - m * m
    scale = g_ref[0] / jnp.sqrt(v + 1.0)
    t = (y_scr[pl.ds(i * BLK, BLK), :] - m) * scale + b_ref[0]
    return jnp.where(t >= 0, t, 0.01 * t)


def _layer_body(x_ref, p_ref, c_ref, a_ref, b_ref, cv_ref, g_ref, be_ref,
                o_ref, y_scr, st_scr):
    ph = pl.program_id(1)
    i = pl.program_id(2)

    @pl.when(ph == 0)
    def _():
        _layer_phase0(i, _layer_compute(x_ref, p_ref, c_ref, a_ref, b_ref,
                                        cv_ref), y_scr, st_scr)

    @pl.when(ph == 1)
    def _():
        o_ref[...] = _bn_lrelu(i, y_scr, st_scr, g_ref, be_ref)


def _layer_fc_body(x_ref, p_ref, c_ref, a_ref, b_ref, cv_ref, g_ref, be_ref,
                   w_ref, fb_ref, o_ref, y_scr, st_scr):
    ph = pl.program_id(1)
    i = pl.program_id(2)

    @pl.when(ph == 0)
    def _():
        _layer_phase0(i, _layer_compute(x_ref, p_ref, c_ref, a_ref, b_ref,
                                        cv_ref), y_scr, st_scr)

    @pl.when(ph == 1)
    def _():
        xn = _bn_lrelu(i, y_scr, st_scr, g_ref, be_ref)
        o_ref[...] = jnp.dot(xn, w_ref[0], precision=HIGH,
                             preferred_element_type=F32) + fb_ref[0]


def _layer_specs():
    return [
        pl.BlockSpec((BLK, H), lambda t, ph, i: (t * NB + i * (1 - ph), 0)),
        pl.BlockSpec((1, BLK, H), lambda t, ph, i: (t, i * (1 - ph), 0)),
        pl.BlockSpec((1, BLK, 16), lambda t, ph, i: (t, i * (1 - ph), 0)),
        pl.BlockSpec((1, H, H), lambda t, ph, i: (t, 0, 0)),
        pl.BlockSpec((1, H, H), lambda t, ph, i: (t, 0, 0)),
        pl.BlockSpec((1, 1, H), lambda t, ph, i: (t, 0, 0)),
        pl.BlockSpec((1, 1, H), lambda t, ph, i: (t, 0, 0)),
        pl.BlockSpec((1, 1, H), lambda t, ph, i: (t, 0, 0)),
    ]


def _out_map(w):
    return lambda t, ph, i: (ph * (t * NB + i) + (1 - ph) * 2 * NB, 0)


def _tc_layer(X, p, cnt, A_l, B_l, cv_l, gamma, beta):
    return pl.pallas_call(
        _layer_body,
        grid=(2, 2, NB),
        in_specs=_layer_specs(),
        out_specs=pl.BlockSpec((BLK, H), _out_map(H)),
        out_shape=jax.ShapeDtypeStruct((2 * N_NODES + BLK, H), F32),
        scratch_shapes=[
            pltpu.VMEM((N_NODES, H), F32),
            pltpu.VMEM((8, H), F32),
        ],
    )(X, p, cnt, A_l, B_l, cv_l, gamma, beta)


def _tc_layer_fc(X, p, cnt, A_l, B_l, cv_l, gamma, beta, fw, fb):
    return pl.pallas_call(
        _layer_fc_body,
        grid=(2, 2, NB),
        in_specs=_layer_specs() + [
            pl.BlockSpec((1, H, 1), lambda t, ph, i: (t, 0, 0)),
            pl.BlockSpec((1, 1, 1), lambda t, ph, i: (t, 0, 0)),
        ],
        out_specs=pl.BlockSpec((BLK, 1), _out_map(1)),
        out_shape=jax.ShapeDtypeStruct((2 * N_NODES + BLK, 1), F32),
        scratch_shapes=[
            pltpu.VMEM((N_NODES, H), F32),
            pltpu.VMEM((8, H), F32),
        ],
    )(X, p, cnt, A_l, B_l, cv_l, gamma, beta, fw, fb.reshape(2, 1, 1))


def _prep_edges(ei, src_off):
    e = ei.shape[1]
    e_pad = NS * N_CHUNKS * CHUNK
    npad = e_pad - e
    ar = jnp.arange(npad, dtype=jnp.int32)
    src = jnp.concatenate([ei[0].astype(jnp.int32) + src_off,
                           ar % (2 * N_NODES)])
    dst = jnp.concatenate([ei[1].astype(jnp.int32),
                           N_NODES + ar % (N_PAD - N_NODES)])
    shape = (NS, N_CHUNKS, CHUNK)
    return (src * 16384 + dst).reshape(shape), dst.reshape(shape)


def kernel(x_user, x_item, edge_index_ui, edge_index_iu, W_src, b_src,
           W_dst, b_dst, W_upd, b_upd, bn_gamma, bn_beta, fc_W, fc_b):
    pk0, d0 = _prep_edges(edge_index_ui, N_NODES)
    pk1, d1 = _prep_edges(edge_index_iu, 0)
    packed = jnp.stack([pk0, pk1])
    dst = jnp.stack([d0, d1])

    cnt = _sc_counts(dst)[:, :, :16]
    A, B, cv = _tc_prep(W_src, W_dst, W_upd, b_src, b_dst, b_upd)
    gam = bn_gamma[:, ::-1].reshape(LAYERS, 2, 1, H)
    bet = bn_beta[:, ::-1].reshape(LAYERS, 2, 1, H)

    X = jnp.concatenate([x_item, x_user], axis=0)
    out = None
    for i in range(LAYERS):
        p = _sc_segsum(X, packed)
        if i < LAYERS - 1:
            X = _tc_layer(X, p, cnt, A[i], B[i], cv[i], gam[i], bet[i])
        else:
            out = _tc_layer_fc(X, p, cnt, A[i], B[i], cv[i], gam[i],
                               bet[i], fc_W[::-1], fc_b[::-1])
    return (out[N_NODES:2 * N_NODES], out[:N_NODES])


# speedup vs baseline: 1.3503x; 1.0340x over previous
import jax
import jax.numpy as jnp
from jax import lax
from jax.experimental import pallas as pl
from jax.experimental.pallas import tpu as pltpu
from jax.experimental.pallas import tpu_sc as plsc

H = 128
N_NODES = 10000
LAYERS = 2
NC = 2
NS = 16
CHUNK = 128
N_CHUNKS = 80
NBUF = 4
N_PAD = 10240
ROWS_PER_TILE = N_PAD // NS
W_CNT = 128
BLK = 1000
NB = N_NODES // BLK
F32 = jnp.float32
HIGH = lax.Precision.HIGHEST


def _zero_acc_stripe(buf, acc_sh, s):
    w = buf.shape[1]

    def zero_body(i, _):
        for k in range(w // 16):
            buf[i, pl.ds(k * 16, 16)] = jnp.zeros((16,), F32)
        return 0
    lax.fori_loop(0, CHUNK, zero_body, 0)
    for k in range(ROWS_PER_TILE // CHUNK):
        pltpu.sync_copy(buf, acc_sh.at[pl.ds(s * ROWS_PER_TILE + k * CHUNK,
                                             CHUNK)])


def _sc_segsum_body(x_hbm, pk_hbm, out_hbm, acc_sh, pk_v, srcb, dstb,
                    r0, r1, s0, s1):
    c = lax.axis_index("c")
    s = lax.axis_index("s")
    rows = (r0, r1)
    sems = (s0, s1)

    _zero_acc_stripe(r0, acc_sh, s)
    plsc.subcore_barrier()

    pltpu.sync_copy(pk_hbm.at[c, s], pk_v)

    def unpack(j, k):
        for g in range(CHUNK // 16):
            pk = pk_v[j, pl.ds(g * 16, 16)]
            srcb[k, pl.ds(g * 16, 16)] = lax.shift_right_logical(pk, 14)
            dstb[k, pl.ds(g * 16, 16)] = lax.bitwise_and(pk, 16383)

    for k in range(2):
        unpack(k, k)
        pltpu.async_copy(x_hbm.at[srcb.at[k]], rows[k], sems[k])

    def body(jj, _):
        for k in range(2):
            j = jj * 2 + k
            pltpu.make_async_copy(x_hbm.at[srcb.at[k]], rows[k],
                                  sems[k]).wait()
            pltpu.sync_copy(rows[k], acc_sh.at[dstb.at[k]], add=True)

            @pl.when(jj < N_CHUNKS // 2 - 1)
            def _():
                unpack(j + 2, k)
                pltpu.async_copy(x_hbm.at[srcb.at[k]], rows[k], sems[k])
        return 0

    lax.fori_loop(0, N_CHUNKS // 2, body, 0)
    plsc.subcore_barrier()

    base = s * ROWS_PER_TILE
    pltpu.sync_copy(acc_sh.at[pl.ds(base, ROWS_PER_TILE)],
                    out_hbm.at[c, pl.ds(base, ROWS_PER_TILE)])


def _sc_segsum(x, packed):
    mesh = plsc.VectorSubcoreMesh(core_axis_name="c", subcore_axis_name="s",
                                  num_cores=NC, num_subcores=NS)
    return pl.kernel(
        _sc_segsum_body,
        out_type=jax.ShapeDtypeStruct((NC, N_PAD, H), F32),
        mesh=mesh,
        scratch_types=[
            pltpu.VMEM_SHARED((N_PAD, H), F32),
            pltpu.VMEM((N_CHUNKS, CHUNK), jnp.int32),
            pltpu.VMEM((2, CHUNK), jnp.int32),
            pltpu.VMEM((2, CHUNK), jnp.int32),
            pltpu.VMEM((CHUNK, H), F32),
            pltpu.VMEM((CHUNK, H), F32),
            pltpu.SemaphoreType.DMA,
            pltpu.SemaphoreType.DMA,
        ],
    )(x, packed)


def _sc_counts_body(dst_hbm, out_hbm, acc_sh, dst_v, ones_v):
    c = lax.axis_index("c")
    s = lax.axis_index("s")

    _zero_acc_stripe(ones_v, acc_sh, s)

    def fill(i, _):
        for k in range(W_CNT // 16):
            ones_v[i, pl.ds(k * 16, 16)] = jnp.ones((16,), F32)
        return 0
    lax.fori_loop(0, CHUNK, fill, 0)
    plsc.subcore_barrier()

    pltpu.sync_copy(dst_hbm.at[c, s], dst_v)

    def body(j, _):
        pltpu.sync_copy(ones_v, acc_sh.at[dst_v.at[j]], add=True)
        return 0
    lax.fori_loop(0, N_CHUNKS, body, 0)
    plsc.subcore_barrier()

    base = s * ROWS_PER_TILE
    pltpu.sync_copy(acc_sh.at[pl.ds(base, ROWS_PER_TILE)],
                    out_hbm.at[c, pl.ds(base, ROWS_PER_TILE)])


def _sc_counts(dst):
    mesh = plsc.VectorSubcoreMesh(core_axis_name="c", subcore_axis_name="s",
                                  num_cores=NC, num_subcores=NS)
    return pl.kernel(
        _sc_counts_body,
        out_type=jax.ShapeDtypeStruct((NC, N_PAD, W_CNT), F32),
        mesh=mesh,
        scratch_types=[
            pltpu.VMEM_SHARED((N_PAD, W_CNT), F32),
            pltpu.VMEM((N_CHUNKS, CHUNK), jnp.int32),
            pltpu.VMEM((CHUNK, W_CNT), F32),
        ],
    )(dst)


def _prep_body(ws_ref, wd_ref, wu_ref, bs_ref, bd_ref, bu_ref,
               a_ref, b_ref, cv_ref):
    wu_top = wu_ref[0, 0, :H, :]
    wu_bot = wu_ref[0, 0, H:, :]
    a_ref[0, 0] = jnp.dot(wd_ref[0, 0], wu_top, precision=HIGH,
                          preferred_element_type=F32)
    b_ref[0, 0] = jnp.dot(ws_ref[0, 0], wu_bot, precision=HIGH,
                          preferred_element_type=F32)
    cv_ref[0, 0] = (jnp.dot(bd_ref[0, 0], wu_top, precision=HIGH,
                            preferred_element_type=F32)
                    + jnp.dot(bs_ref[0, 0], wu_bot, precision=HIGH,
                              preferred_element_type=F32)
                    + bu_ref[0, 0])


def _tc_prep(W_src, W_dst, W_upd, b_src, b_dst, b_upd):
    bs = b_src.reshape(LAYERS, 2, 1, H)
    bd = b_dst.reshape(LAYERS, 2, 1, H)
    bu = b_upd.reshape(LAYERS, 2, 1, H)
    m4 = lambda i, j: (i, j, 0, 0)
    return pl.pallas_call(
        _prep_body,
        grid=(LAYERS, 2),
        in_specs=[
            pl.BlockSpec((1, 1, H, H), m4),
            pl.BlockSpec((1, 1, H, H), m4),
            pl.BlockSpec((1, 1, 2 * H, H), m4),
            pl.BlockSpec((1, 1, 1, H), m4),
            pl.BlockSpec((1, 1, 1, H), m4),
            pl.BlockSpec((1, 1, 1, H), m4),
        ],
        out_specs=[
            pl.BlockSpec((1, 1, H, H), m4),
            pl.BlockSpec((1, 1, H, H), m4),
            pl.BlockSpec((1, 1, 1, H), m4),
        ],
        out_shape=[
            jax.ShapeDtypeStruct((LAYERS, 2, H, H), F32),
            jax.ShapeDtypeStruct((LAYERS, 2, H, H), F32),
            jax.ShapeDtypeStruct((LAYERS, 2, 1, H), F32),
        ],
    )(W_src, W_dst, W_upd, bs, bd, bu)


def _layer_compute(x_ref, p_ref, c_ref, a_ref, b_ref, cv_ref):
    cnt = c_ref[0, :, 0:1]
    recip = 1.0 / jnp.maximum(cnt, 1.0)
    agg = p_ref[0] * recip
    return (jnp.dot(x_ref[...], a_ref[0], preferred_element_type=F32)
            + jnp.dot(agg, b_ref[0], preferred_element_type=F32)
            + cv_ref[0])


def _layer_phase0(i, y, y_scr, st_scr):
    y_scr[pl.ds(i * BLK, BLK), :] = y

    @pl.when(i == 0)
    def _():
        st_scr[...] = jnp.zeros_like(st_scr)

    sums = jnp.sum(y, axis=0)[None, :]
    sumsq = jnp.sum(y * y, axis=0)[None, :]
    st_scr[...] += jnp.concatenate(
        [sums, sumsq, jnp.zeros((6, H), F32)], axis=0)


def _bn_lrelu(i, y_scr, st_scr, g_ref, b_ref):
    n = jnp.float32(N_NODES)
    m = st_scr[0:1, :] / n
    v = st_scr[1:2, :] / n - m * m
    scale = g_ref[0] / jnp.sqrt(v + 1.0)
    t = (y_scr[pl.ds(i * BLK, BLK), :] - m) * scale + b_ref[0]
    return jnp.where(t >= 0, t, 0.01 * t)


def _layer_body(x_ref, p_ref, c_ref, a_ref, b_ref, cv_ref, g_ref, be_ref,
                o_ref, y_scr, st_scr):
    ph = pl.program_id(1)
    i = pl.program_id(2)

    @pl.when(ph == 0)
    def _():
        _layer_phase0(i, _layer_compute(x_ref, p_ref, c_ref, a_ref, b_ref,
                                        cv_ref), y_scr, st_scr)

    @pl.when(ph == 1)
    def _():
        o_ref[...] = _bn_lrelu(i, y_scr, st_scr, g_ref, be_ref)


def _layer_fc_body(x_ref, p_ref, c_ref, a_ref, b_ref, cv_ref, g_ref, be_ref,
                   w_ref, fb_ref, o_ref, y_scr, st_scr):
    ph = pl.program_id(1)
    i = pl.program_id(2)

    @pl.when(ph == 0)
    def _():
        _layer_phase0(i, _layer_compute(x_ref, p_ref, c_ref, a_ref, b_ref,
                                        cv_ref), y_scr, st_scr)

    @pl.when(ph == 1)
    def _():
        xn = _bn_lrelu(i, y_scr, st_scr, g_ref, be_ref)
        o_ref[...] = jnp.dot(xn, w_ref[0],
                             preferred_element_type=F32) + fb_ref[0]


def _layer_specs():
    return [
        pl.BlockSpec((BLK, H), lambda t, ph, i: (t * NB + i * (1 - ph), 0)),
        pl.BlockSpec((1, BLK, H), lambda t, ph, i: (t, i * (1 - ph), 0)),
        pl.BlockSpec((1, BLK, 16), lambda t, ph, i: (t, i * (1 - ph), 0)),
        pl.BlockSpec((1, H, H), lambda t, ph, i: (t, 0, 0)),
        pl.BlockSpec((1, H, H), lambda t, ph, i: (t, 0, 0)),
        pl.BlockSpec((1, 1, H), lambda t, ph, i: (t, 0, 0)),
        pl.BlockSpec((1, 1, H), lambda t, ph, i: (t, 0, 0)),
        pl.BlockSpec((1, 1, H), lambda t, ph, i: (t, 0, 0)),
    ]


def _out_map(w):
    return lambda t, ph, i: (ph * (t * NB + i) + (1 - ph) * 2 * NB, 0)


def _tc_layer(X, p, cnt, A_l, B_l, cv_l, gamma, beta):
    return pl.pallas_call(
        _layer_body,
        grid=(2, 2, NB),
        in_specs=_layer_specs(),
        out_specs=pl.BlockSpec((BLK, H), _out_map(H)),
        out_shape=jax.ShapeDtypeStruct((2 * N_NODES + BLK, H), F32),
        scratch_shapes=[
            pltpu.VMEM((N_NODES, H), F32),
            pltpu.VMEM((8, H), F32),
        ],
    )(X, p, cnt, A_l, B_l, cv_l, gamma, beta)


def _tc_layer_fc(X, p, cnt, A_l, B_l, cv_l, gamma, beta, fw, fb):
    return pl.pallas_call(
        _layer_fc_body,
        grid=(2, 2, NB),
        in_specs=_layer_specs() + [
            pl.BlockSpec((1, H, 1), lambda t, ph, i: (t, 0, 0)),
            pl.BlockSpec((1, 1, 1), lambda t, ph, i: (t, 0, 0)),
        ],
        out_specs=pl.BlockSpec((BLK, 1), _out_map(1)),
        out_shape=jax.ShapeDtypeStruct((2 * N_NODES + BLK, 1), F32),
        scratch_shapes=[
            pltpu.VMEM((N_NODES, H), F32),
            pltpu.VMEM((8, H), F32),
        ],
    )(X, p, cnt, A_l, B_l, cv_l, gamma, beta, fw, fb.reshape(2, 1, 1))


def _prep_edges(ei, src_off):
    e = ei.shape[1]
    e_pad = NS * N_CHUNKS * CHUNK
    npad = e_pad - e
    ar = jnp.arange(npad, dtype=jnp.int32)
    src = jnp.concatenate([ei[0].astype(jnp.int32) + src_off,
                           ar % (2 * N_NODES)])
    dst = jnp.concatenate([ei[1].astype(jnp.int32),
                           N_NODES + ar % (N_PAD - N_NODES)])
    shape = (NS, N_CHUNKS, CHUNK)
    return (src * 16384 + dst).reshape(shape), dst.reshape(shape)


def kernel(x_user, x_item, edge_index_ui, edge_index_iu, W_src, b_src,
           W_dst, b_dst, W_upd, b_upd, bn_gamma, bn_beta, fc_W, fc_b):
    pk0, d0 = _prep_edges(edge_index_ui, N_NODES)
    pk1, d1 = _prep_edges(edge_index_iu, 0)
    dst = jnp.stack([d0, d1])

    cnt = _sc_counts(dst)[:, :, :16]
    packed = jnp.stack([pk0, pk1]) + (cnt[0, 0, 0:1] * 0.0).astype(jnp.int32)
    A, B, cv = _tc_prep(W_src, W_dst, W_upd, b_src, b_dst, b_upd)
    gam = bn_gamma[:, ::-1].reshape(LAYERS, 2, 1, H)
    bet = bn_beta[:, ::-1].reshape(LAYERS, 2, 1, H)

    X = jnp.concatenate([x_item, x_user], axis=0)
    out = None
    for i in range(LAYERS):
        p = _sc_segsum(X, packed)
        if i < LAYERS - 1:
            X = _tc_layer(X, p, cnt, A[i], B[i], cv[i], gam[i], bet[i])
        else:
            out = _tc_layer_fc(X, p, cnt, A[i], B[i], cv[i], gam[i],
                               bet[i], fc_W[::-1], fc_b[::-1])
    return (out[N_NODES:2 * N_NODES], out[:N_NODES])
